# baseline (speedup 1.0000x reference)
import numpy as np

import jax
import jax.numpy as jnp
from jax.experimental import pallas as pl
from jax.experimental.pallas import tpu as pltpu
from jax.experimental.shard_map import shard_map
from jax.sharding import Mesh, PartitionSpec as P

N = 10000
E = 256
EH = E // 2
C = 256
HEADS = 8
TILE = 2000
GRID = N // TILE
NCP = 128

f32 = jnp.float32
bf16 = jnp.bfloat16


def _dot(a, b):
    return jax.lax.dot_general(a, b, (((1,), (0,)), ((), ())),
                               preferred_element_type=f32)


def _dot_tn(a, b):
    return jax.lax.dot_general(a, b, (((0,), (0,)), ((), ())),
                               preferred_element_type=f32)


def _mega(dev_ref, x_ref, h_ref, w1_ref, b1_ref, att1_ref, att1t_ref,
          w11_ref, b11_ref, w2_ref, b2_ref, att2_ref, att2t_ref, w21_ref,
          b21_ref,
          klv1_ref, klv2_ref, y1_ref, y2_ref,
          hb_s, x1_s, x2_s, s_s, ze_s, t_s, de_s, dec_s, idv_s, yb_s, ub_s,
          zth_s):
    p = pl.program_id(0)
    i = pl.program_id(1)
    rows = pl.ds(i * TILE, TILE)
    eoff = pl.multiple_of(dev_ref[0] * EH, EH)

    def _finalize_edge(att_ref):
        zeh = ze_s[pl.ds(eoff, EH), :]
        attp = att_ref[...]
        zn2 = _dot(zeh * zeh, jnp.transpose(attp))
        inv_zn = jax.lax.rsqrt(jnp.maximum(zn2, 1e-24))
        for hh in range(HEADS):
            zth_s[hh * C:(hh + 1) * C, :] = jnp.transpose(
                (zeh * attp[hh:hh + 1, :] * inv_zn[:, hh:hh + 1])
            ).astype(bf16)

    def _attn_kl(xb, attt_ref):
        xn2 = _dot(xb * xb, attt_ref[...])
        inv_xn = (jax.lax.rsqrt(jnp.maximum(xn2, 1e-24)) *
                  (1.0 / HEADS)).astype(bf16)
        acc = jnp.zeros((TILE, EH), f32)
        for hh in range(HEADS):
            acc += _dot(xb * inv_xn[:, hh:hh + 1],
                        zth_s[hh * C:(hh + 1) * C, :])
        ac = jnp.minimum(jnp.maximum(acc, 1e-6), 1.0 - 1e-6)
        kl = ac * jnp.log(ac * 2.0) + (1.0 - ac) * jnp.log((1.0 - ac) * 2.0)
        return jnp.sum(kl, axis=0, keepdims=True)

    @pl.when(jnp.logical_and(p == 0, i == 0))
    def _():
        s_s[...] = jnp.zeros_like(s_s)
        de_s[...] = jnp.zeros_like(de_s)
        klv1_ref[...] = jnp.zeros_like(klv1_ref)
        klv2_ref[...] = jnp.zeros_like(klv2_ref)

    @pl.when(p == 0)
    def _():
        h = h_ref[...]
        hb = h.astype(bf16)
        hb_s[rows, :] = hb
        idv_s[rows, :] = 1.0 / jnp.maximum(
            jnp.sum(h, axis=1, keepdims=True), 1.0)
        xw = _dot(x_ref[...].astype(bf16), w1_ref[...]) + b1_ref[...]
        s_s[...] += _dot_tn(hb, xw.astype(bf16))
        de_s[...] += jnp.sum(h, axis=0, keepdims=True)

    @pl.when(jnp.logical_and(p == 0, i == GRID - 1))
    def _():
        dec = jnp.maximum(de_s[...], 1.0).reshape(E, 1)
        dec_s[...] = dec
        yb_s[...] = (s_s[...] / dec).astype(bf16)

    @pl.when(jnp.logical_and(p == 1, i == 0))
    def _():
        ze_s[...] = jnp.zeros_like(ze_s)

    @pl.when(p == 1)
    def _():
        hb = hb_s[rows, :]
        x1 = jnp.maximum(_dot(hb, yb_s[...]) * idv_s[rows, :],
                         0.0).astype(bf16)
        x1_s[rows, :] = x1
        ze_s[...] += _dot_tn(hb, x1)

    @pl.when(jnp.logical_and(p == 1, i == GRID - 1))
    def _():
        _finalize_edge(att1_ref)

    @pl.when(jnp.logical_and(p == 2, i == 0))
    def _():
        s_s[...] = jnp.zeros_like(s_s)
        t_s[...] = jnp.zeros_like(t_s)

    @pl.when(p == 2)
    def _():
        x1 = x1_s[rows, :]
        hb = hb_s[rows, :]
        klv1_ref[...] += _attn_kl(x1, att1t_ref)
        t_s[...] += _dot_tn(hb, (_dot(x1, w11_ref[...]) +
                                 b11_ref[...]).astype(bf16))
        s_s[...] += _dot_tn(hb, (_dot(x1, w2_ref[...]) +
                                 b2_ref[...]).astype(bf16))

    @pl.when(jnp.logical_and(p == 2, i == GRID - 1))
    def _():
        dec = dec_s[...]
        yb_s[...] = (s_s[...] / dec).astype(bf16)
        ub_s[...] = (t_s[...] / dec).astype(bf16)

    @pl.when(jnp.logical_and(p == 3, i == 0))
    def _():
        ze_s[...] = jnp.zeros_like(ze_s)

    @pl.when(p == 3)
    def _():
        hb = hb_s[rows, :]
        idv = idv_s[rows, :]
        x2 = jnp.maximum(_dot(hb, yb_s[...]) * idv, 0.0).astype(bf16)
        x2_s[rows, :] = x2
        y1_ref[...] = _dot(hb, ub_s[...]) * idv
        ze_s[...] += _dot_tn(hb, x2)

    @pl.when(jnp.logical_and(p == 3, i == GRID - 1))
    def _():
        _finalize_edge(att2_ref)

    @pl.when(jnp.logical_and(p == 4, i == 0))
    def _():
        t_s[...] = jnp.zeros_like(t_s)

    @pl.when(p == 4)
    def _():
        x2 = x2_s[rows, :]
        hb = hb_s[rows, :]
        klv2_ref[...] += _attn_kl(x2, att2t_ref)
        t_s[...] += _dot_tn(hb, (_dot(x2, w21_ref[...]) +
                                 b21_ref[...]).astype(bf16))

    @pl.when(jnp.logical_and(p == 4, i == GRID - 1))
    def _():
        ub_s[...] = (t_s[...] / dec_s[...]).astype(bf16)

    @pl.when(p == 5)
    def _():
        hb = hb_s[rows, :]
        y2_ref[...] = _dot(hb, ub_s[...]) * idv_s[rows, :]


def _stream_spec(cols, phase):
    def idx(p, i):
        return (jnp.where(p == phase, i, jnp.where(p < phase, 0, GRID - 1)),
                0)
    return pl.BlockSpec((TILE, cols), idx)


def _const_spec(rows, cols):
    return pl.BlockSpec((rows, cols), lambda p, i: (0, 0))


def _pipeline(dev, x, H, w1b, b1r, att1sq, att1sqT, w11p, b11p, w2b, b2r,
              att2sq, att2sqT, w21p, b21p):
    return pl.pallas_call(
        _mega,
        grid=(6, GRID),
        in_specs=[
            pl.BlockSpec(memory_space=pltpu.SMEM),
            _stream_spec(C, 0),
            _stream_spec(E, 0),
            _const_spec(C, C),
            _const_spec(1, C),
            _const_spec(NCP, C),
            _const_spec(C, NCP),
            _const_spec(C, NCP),
            _const_spec(1, NCP),
            _const_spec(C, C),
            _const_spec(1, C),
            _const_spec(NCP, C),
            _const_spec(C, NCP),
            _const_spec(C, NCP),
            _const_spec(1, NCP),
        ],
        out_specs=[
            _const_spec(1, EH),
            _const_spec(1, EH),
            _stream_spec(NCP, 3),
            _stream_spec(NCP, 5),
        ],
        out_shape=[
            jax.ShapeDtypeStruct((1, EH), f32),
            jax.ShapeDtypeStruct((1, EH), f32),
            jax.ShapeDtypeStruct((N, NCP), f32),
            jax.ShapeDtypeStruct((N, NCP), f32),
        ],
        scratch_shapes=[
            pltpu.VMEM((N, E), bf16),
            pltpu.VMEM((N, C), bf16),
            pltpu.VMEM((N, C), bf16),
            pltpu.VMEM((E, C), f32),
            pltpu.VMEM((E, C), f32),
            pltpu.VMEM((E, NCP), f32),
            pltpu.VMEM((1, E), f32),
            pltpu.VMEM((E, 1), f32),
            pltpu.VMEM((N, 1), f32),
            pltpu.VMEM((E, C), bf16),
            pltpu.VMEM((E, NCP), bf16),
            pltpu.VMEM((HEADS * C, EH), bf16),
        ],
        compiler_params=pltpu.CompilerParams(
            dimension_semantics=("arbitrary", "arbitrary")),
    )(dev, x, H, w1b, b1r, att1sq, att1sqT, w11p, b11p,
      w2b, b2r, att2sq, att2sqT, w21p, b21p)


def kernel(x, H, W1, b1, att1, W11, b11, W2, b2, att2, W21, b21):
    b1r = b1.reshape(1, C)
    b2r = b2.reshape(1, C)
    att1sq = jnp.zeros((NCP, C), f32).at[:HEADS].set(att1 * att1)
    att2sq = jnp.zeros((NCP, C), f32).at[:HEADS].set(att2 * att2)
    att1sqT = att1sq.T.astype(bf16)
    att2sqT = att2sq.T.astype(bf16)
    w11p = jnp.zeros((C, NCP), bf16).at[:, :3].set(W11.astype(bf16))
    b11p = jnp.zeros((1, NCP), f32).at[0, :3].set(b11)
    w21p = jnp.zeros((C, NCP), bf16).at[:, :3].set(W21.astype(bf16))
    b21p = jnp.zeros((1, NCP), f32).at[0, :3].set(b21)
    w1b = W1.astype(bf16)
    w2b = W2.astype(bf16)
    args = (x, H, w1b, b1r, att1sq, att1sqT, w11p, b11p,
            w2b, b2r, att2sq, att2sqT, w21p, b21p)

    devs = jax.devices()
    if len(devs) >= 2:
        mesh = Mesh(np.array(devs[:2]), ("d",))

        def _sharded(*a):
            dev = jax.lax.axis_index("d").reshape(1).astype(jnp.int32)
            klv1, klv2, y1p, y2p = _pipeline(dev, *a)
            kl1 = jax.lax.psum(jnp.sum(klv1), "d") / N
            kl2 = jax.lax.psum(jnp.sum(klv2), "d") / N
            return y1p, y2p, (kl1 + kl2) * 0.5

        y1p, y2p, kl = shard_map(
            _sharded, mesh=mesh,
            in_specs=tuple(P() for _ in args),
            out_specs=(P(), P(), P()),
            check_rep=False,
        )(*args)
    else:
        dev = jnp.zeros((1,), jnp.int32)
        klv1a, klv2a, y1p, y2p = _pipeline(dev, *args)
        dev1 = jnp.ones((1,), jnp.int32)
        klv1b, klv2b, _, _ = _pipeline(dev1, *args)
        kl1 = (jnp.sum(klv1a) + jnp.sum(klv1b)) / N
        kl2 = (jnp.sum(klv2a) + jnp.sum(klv2b)) / N
        kl = (kl1 + kl2) * 0.5

    return (y1p[:, :3], y2p[:, :3], kl)


# speedup vs baseline: 2.1422x; 2.1422x over previous
import jax
import jax.numpy as jnp
from jax.experimental import pallas as pl
from jax.experimental.pallas import tpu as pltpu

N = 10000
E = 256
C = 256
HEADS = 8
TILE = 2000
GRID = N // TILE
NCP = 128

f32 = jnp.float32
bf16 = jnp.bfloat16


def _dot(a, b):
    return jax.lax.dot_general(a, b, (((1,), (0,)), ((), ())),
                               preferred_element_type=f32)


def _dot_tn(a, b):
    return jax.lax.dot_general(a, b, (((0,), (0,)), ((), ())),
                               preferred_element_type=f32)


def _mega(x_ref, h_ref, w1_ref, b1_ref, att1_ref, att1t_ref, w11_ref,
          b11_ref, w2_ref, b2_ref, att2_ref, att2t_ref, w21_ref, b21_ref,
          klv1_ref, klv2_ref, y1_ref, y2_ref,
          hb_s, x1_s, x2_s, s_s, ze_s, t_s, de_s, dec_s, idv_s, yb_s, ub_s,
          zth_s):
    p = pl.program_id(0)
    i = pl.program_id(1)
    rows = pl.ds(i * TILE, TILE)

    def _finalize_edge(ze, att_ref):
        attp = att_ref[...]
        zn2 = _dot(ze * ze, jnp.transpose(attp))
        inv_zn = jax.lax.rsqrt(jnp.maximum(zn2, 1e-24))
        for hh in range(HEADS):
            zth_s[hh * C:(hh + 1) * C, :] = jnp.transpose(
                (ze * attp[hh:hh + 1, :] * inv_zn[:, hh:hh + 1])
            ).astype(bf16)

    def _attn_kl(xb, attt_ref):
        xn2 = _dot(xb * xb, attt_ref[...])
        inv_xn = (jax.lax.rsqrt(jnp.maximum(xn2, 1e-24)) *
                  (1.0 / HEADS)).astype(bf16)
        acc = jnp.zeros((TILE, E), f32)
        for hh in range(HEADS):
            acc += _dot(xb * inv_xn[:, hh:hh + 1],
                        zth_s[hh * C:(hh + 1) * C, :])
        ac = jnp.minimum(jnp.maximum(acc, 1e-6), 1.0 - 1e-6)
        kl = ac * jnp.log(ac * 2.0) + (1.0 - ac) * jnp.log((1.0 - ac) * 2.0)
        return jnp.sum(kl, axis=0, keepdims=True)

    @pl.when(jnp.logical_and(p == 0, i == 0))
    def _():
        s_s[...] = jnp.zeros_like(s_s)
        de_s[...] = jnp.zeros_like(de_s)
        klv1_ref[...] = jnp.zeros_like(klv1_ref)
        klv2_ref[...] = jnp.zeros_like(klv2_ref)

    @pl.when(p == 0)
    def _():
        h = h_ref[...]
        hb = h.astype(bf16)
        hb_s[rows, :] = hb
        idv_s[rows, :] = 1.0 / jnp.maximum(
            jnp.sum(h, axis=1, keepdims=True), 1.0)
        xw = _dot(x_ref[...].astype(bf16), w1_ref[...]) + b1_ref[...]
        s_s[...] += _dot_tn(hb, xw.astype(bf16))
        de_s[...] += jnp.sum(h, axis=0, keepdims=True)

    @pl.when(jnp.logical_and(p == 0, i == GRID - 1))
    def _():
        dec = jnp.maximum(de_s[...], 1.0).reshape(E, 1)
        dec_s[...] = dec
        yb_s[...] = (s_s[...] / dec).astype(bf16)

    @pl.when(jnp.logical_and(p == 1, i == 0))
    def _():
        ze_s[...] = jnp.zeros_like(ze_s)

    @pl.when(p == 1)
    def _():
        hb = hb_s[rows, :]
        x1 = jnp.maximum(_dot(hb, yb_s[...]) * idv_s[rows, :],
                         0.0).astype(bf16)
        x1_s[rows, :] = x1
        ze_s[...] += _dot_tn(hb, x1)

    @pl.when(jnp.logical_and(p == 1, i == GRID - 1))
    def _():
        _finalize_edge(ze_s[...], att1_ref)

    @pl.when(jnp.logical_and(p == 2, i == 0))
    def _():
        s_s[...] = jnp.zeros_like(s_s)
        t_s[...] = jnp.zeros_like(t_s)

    @pl.when(p == 2)
    def _():
        x1 = x1_s[rows, :]
        hb = hb_s[rows, :]
        klv1_ref[...] += _attn_kl(x1, att1t_ref)
        t_s[...] += _dot_tn(hb, (_dot(x1, w11_ref[...]) +
                                 b11_ref[...]).astype(bf16))
        s_s[...] += _dot_tn(hb, (_dot(x1, w2_ref[...]) +
                                 b2_ref[...]).astype(bf16))

    @pl.when(jnp.logical_and(p == 2, i == GRID - 1))
    def _():
        dec = dec_s[...]
        yb_s[...] = (s_s[...] / dec).astype(bf16)
        ub_s[...] = (t_s[...] / dec).astype(bf16)

    @pl.when(jnp.logical_and(p == 3, i == 0))
    def _():
        ze_s[...] = jnp.zeros_like(ze_s)

    @pl.when(p == 3)
    def _():
        hb = hb_s[rows, :]
        idv = idv_s[rows, :]
        x2 = jnp.maximum(_dot(hb, yb_s[...]) * idv, 0.0).astype(bf16)
        x2_s[rows, :] = x2
        y1_ref[...] = _dot(hb, ub_s[...]) * idv
        ze_s[...] += _dot_tn(hb, x2)

    @pl.when(jnp.logical_and(p == 3, i == GRID - 1))
    def _():
        _finalize_edge(ze_s[...], att2_ref)

    @pl.when(jnp.logical_and(p == 4, i == 0))
    def _():
        t_s[...] = jnp.zeros_like(t_s)

    @pl.when(p == 4)
    def _():
        x2 = x2_s[rows, :]
        hb = hb_s[rows, :]
        klv2_ref[...] += _attn_kl(x2, att2t_ref)
        t_s[...] += _dot_tn(hb, (_dot(x2, w21_ref[...]) +
                                 b21_ref[...]).astype(bf16))

    @pl.when(jnp.logical_and(p == 4, i == GRID - 1))
    def _():
        ub_s[...] = (t_s[...] / dec_s[...]).astype(bf16)

    @pl.when(p == 5)
    def _():
        hb = hb_s[rows, :]
        y2_ref[...] = _dot(hb, ub_s[...]) * idv_s[rows, :]


def _stream_spec(cols, phase):
    def idx(p, i):
        return (jnp.where(p == phase, i, jnp.where(p < phase, 0, GRID - 1)),
                0)
    return pl.BlockSpec((TILE, cols), idx)


def _const_spec(rows, cols):
    return pl.BlockSpec((rows, cols), lambda p, i: (0, 0))


def kernel(x, H, W1, b1, att1, W11, b11, W2, b2, att2, W21, b21):
    b1r = b1.reshape(1, C)
    b2r = b2.reshape(1, C)
    att1sq = jnp.zeros((NCP, C), f32).at[:HEADS].set(att1 * att1)
    att2sq = jnp.zeros((NCP, C), f32).at[:HEADS].set(att2 * att2)
    att1sqT = att1sq.T.astype(bf16)
    att2sqT = att2sq.T.astype(bf16)
    w11p = jnp.zeros((C, NCP), bf16).at[:, :3].set(W11.astype(bf16))
    b11p = jnp.zeros((1, NCP), f32).at[0, :3].set(b11)
    w21p = jnp.zeros((C, NCP), bf16).at[:, :3].set(W21.astype(bf16))
    b21p = jnp.zeros((1, NCP), f32).at[0, :3].set(b21)

    klv1, klv2, y1p, y2p = pl.pallas_call(
        _mega,
        grid=(6, GRID),
        in_specs=[
            _stream_spec(C, 0),
            _stream_spec(E, 0),
            _const_spec(C, C),
            _const_spec(1, C),
            _const_spec(NCP, C),
            _const_spec(C, NCP),
            _const_spec(C, NCP),
            _const_spec(1, NCP),
            _const_spec(C, C),
            _const_spec(1, C),
            _const_spec(NCP, C),
            _const_spec(C, NCP),
            _const_spec(C, NCP),
            _const_spec(1, NCP),
        ],
        out_specs=[
            _const_spec(1, E),
            _const_spec(1, E),
            _stream_spec(NCP, 3),
            _stream_spec(NCP, 5),
        ],
        out_shape=[
            jax.ShapeDtypeStruct((1, E), f32),
            jax.ShapeDtypeStruct((1, E), f32),
            jax.ShapeDtypeStruct((N, NCP), f32),
            jax.ShapeDtypeStruct((N, NCP), f32),
        ],
        scratch_shapes=[
            pltpu.VMEM((N, E), bf16),
            pltpu.VMEM((N, C), bf16),
            pltpu.VMEM((N, C), bf16),
            pltpu.VMEM((E, C), f32),
            pltpu.VMEM((E, C), f32),
            pltpu.VMEM((E, NCP), f32),
            pltpu.VMEM((1, E), f32),
            pltpu.VMEM((E, 1), f32),
            pltpu.VMEM((N, 1), f32),
            pltpu.VMEM((E, C), bf16),
            pltpu.VMEM((E, NCP), bf16),
            pltpu.VMEM((HEADS * C, E), bf16),
        ],
        compiler_params=pltpu.CompilerParams(
            dimension_semantics=("arbitrary", "arbitrary")),
    )(x, H, W1.astype(bf16), b1r, att1sq, att1sqT, w11p, b11p,
      W2.astype(bf16), b2r, att2sq, att2sqT, w21p, b21p)

    kl1 = jnp.sum(klv1) / N
    kl2 = jnp.sum(klv2) / N
    return (y1p[:, :3], y2p[:, :3], (kl1 + kl2) * 0.5)
